# baseline (speedup 1.0000x reference)
import functools

import jax
import jax.numpy as jnp
from jax import lax
from jax.experimental import pallas as pl
from jax.experimental.pallas import tpu as pltpu
from jax.experimental.pallas import tpu_sc as plsc

VOCAB = 100000
MODEL_DIM = 128
BATCH = 1024
SEQ = 200

NC, NS = 2, 16
NW = NC * NS
ROWS = BATCH * SEQ
ROWS_PER_W = ROWS // NW
CHUNK = 128
NCHUNK = ROWS_PER_W // CHUNK
NBUF = 4
RING = (NCHUNK // NBUF) * NBUF


def _pos_table():
    pos = jnp.arange(SEQ, dtype=jnp.float32)[:, None]
    i = jnp.arange(MODEL_DIM)[None, :]
    angle = pos / jnp.power(10000.0, (2 * (i // 2)).astype(jnp.float32) / float(MODEL_DIM))
    sin_v = jnp.sin(angle[:, 0::2])
    cos_v = jnp.cos(angle[:, 1::2])
    pe = jnp.concatenate([sin_v[..., None], cos_v[..., None]], axis=-1)
    return pe.reshape(SEQ, MODEL_DIM)


def _sc_body(idx_hbm, pe_hbm, table_hbm, out_hbm, idx_v, pe_sh, stage_sh,
             ob0, ob1, ob2, ob3, gs0, gs1, gs2, gs3,
             cs0, cs1, hs0, hs1, ps0, ps1, ps2, ps3):
    sid = lax.axis_index("s")
    wid = sid * NC + lax.axis_index("c")
    pltpu.sync_copy(idx_hbm.at[wid], idx_v)

    @pl.when(sid == 0)
    def _():
        pltpu.sync_copy(pe_hbm, pe_sh)

    plsc.subcore_barrier()

    base = wid * ROWS_PER_W
    obs = (ob0, ob1, ob2, ob3)
    gsems = (gs0, gs1, gs2, gs3)
    csems = (cs0, cs1)
    hsems = (hs0, hs1)
    psems = (ps0, ps1, ps2, ps3)

    def pe_slice(cc):
        return pe_sh.at[pl.ds((cc * CHUNK) % SEQ, CHUNK)]

    def pe_fill(cc, bb):
        pltpu.async_copy(pe_slice(cc), obs[bb], psems[bb])

    def gather_add(cc, bb):
        pltpu.async_copy(table_hbm.at[idx_v.at[cc]], obs[bb], gsems[bb],
                         add=True)

    def stage_slice(ss):
        return stage_sh.at[pl.ds((sid * 2 + ss) * CHUNK, CHUNK)]

    def out_slice(cc):
        return out_hbm.at[pl.ds(base + cc * CHUNK, CHUNK)]

    def turn(cc, b):
        o_v = obs[b]
        s = b % 2
        pltpu.make_async_copy(table_hbm.at[idx_v.at[cc]], o_v,
                              gsems[b]).wait()

        @pl.when(cc >= 2)
        def _():
            pltpu.make_async_copy(stage_slice(s), out_slice(cc),
                                  hsems[s]).wait()

        pltpu.async_copy(o_v, stage_slice(s), csems[s])

        @pl.when(cc >= 1)
        def _():
            b1 = (b - 1) % NBUF
            s1 = (b - 1) % 2
            pltpu.make_async_copy(obs[b1], stage_slice(s1),
                                  csems[s1]).wait()
            pltpu.async_copy(stage_slice(s1), out_slice(cc - 1),
                             hsems[s1])

            @pl.when(cc + NBUF - 1 < NCHUNK)
            def _():
                pe_fill(cc + NBUF - 1, b1)

        @pl.when(cc + 2 < NCHUNK)
        def _():
            b2 = (b + 2) % NBUF
            pltpu.make_async_copy(pe_slice(cc), obs[b2],
                                  psems[b2]).wait()
            gather_add(cc + 2, b2)

    for b in range(NBUF):
        pe_fill(b, b)
    for b in range(2):
        pltpu.make_async_copy(pe_slice(b), obs[b], psems[b]).wait()
        gather_add(b, b)

    def ring_body(i, carry):
        c = i * NBUF
        for b in range(NBUF):
            turn(c + b, b)
        return carry

    lax.fori_loop(0, RING // NBUF, ring_body, 0)
    for cc in range(RING, NCHUNK):
        turn(jnp.int32(cc), cc % NBUF)
    last_b = (NCHUNK - 1) % NBUF
    last_s = last_b % 2
    pltpu.make_async_copy(obs[last_b], stage_slice(last_s),
                          csems[last_s]).wait()
    pltpu.async_copy(stage_slice(last_s), out_slice(NCHUNK - 1),
                     hsems[last_s])
    for s in range(2):
        pltpu.make_async_copy(stage_slice(s), out_hbm.at[pl.ds(base, CHUNK)],
                              hsems[s]).wait()


@jax.jit
def kernel(inputs, tok_emb_table):
    idx3 = inputs.reshape(NW, NCHUNK, CHUNK).astype(jnp.int32)
    pe1 = _pos_table()
    pe = jnp.concatenate([pe1, pe1], axis=0)
    mesh = plsc.VectorSubcoreMesh(core_axis_name="c", subcore_axis_name="s")
    run = functools.partial(
        pl.kernel,
        mesh=mesh,
        out_type=jax.ShapeDtypeStruct((ROWS, MODEL_DIM), jnp.float32),
        scratch_types=[
            pltpu.VMEM((NCHUNK, CHUNK), jnp.int32),
            pltpu.VMEM_SHARED((2 * SEQ, MODEL_DIM), jnp.float32),
            pltpu.VMEM_SHARED((NS * 2 * CHUNK, MODEL_DIM), jnp.float32),
        ] + [pltpu.VMEM((CHUNK, MODEL_DIM), jnp.float32)] * NBUF
          + [pltpu.SemaphoreType.DMA] * (2 * NBUF + 4),
    )(_sc_body)
    out = run(idx3, pe, tok_emb_table)
    return out.reshape(BATCH, SEQ, MODEL_DIM)


# speedup vs baseline: 1.1059x; 1.1059x over previous
import functools

import jax
import jax.numpy as jnp
from jax import lax
from jax.experimental import pallas as pl
from jax.experimental.pallas import tpu as pltpu
from jax.experimental.pallas import tpu_sc as plsc

VOCAB = 100000
MODEL_DIM = 128
BATCH = 1024
SEQ = 200

NC, NS = 2, 16
NW = NC * NS
ROWS = BATCH * SEQ
ROWS_PER_W = ROWS // NW
CHUNK = 128
NCHUNK = ROWS_PER_W // CHUNK
NBUF = 5
SKEW = 4


def _pos_table():
    pos = jnp.arange(SEQ, dtype=jnp.float32)[:, None]
    i = jnp.arange(MODEL_DIM)[None, :]
    angle = pos / jnp.power(10000.0, (2 * (i // 2)).astype(jnp.float32) / float(MODEL_DIM))
    sin_v = jnp.sin(angle[:, 0::2])
    cos_v = jnp.cos(angle[:, 1::2])
    pe = jnp.concatenate([sin_v[..., None], cos_v[..., None]], axis=-1)
    return pe.reshape(SEQ, MODEL_DIM)


def _sc_body(idx_hbm, pe_hbm, table_hbm, out_hbm, idx_v, pe_sh,
             ob0, ob1, ob2, ob3, ob4, gs0, gs1, gs2, gs3, gs4,
             ss0, ss1, ss2, ss3, ss4, ps0, ps1, ps2, ps3, ps4):
    sid = lax.axis_index("s")
    wid = sid * NC + lax.axis_index("c")
    pltpu.sync_copy(idx_hbm.at[wid], idx_v)

    @pl.when(sid == 0)
    def _():
        pltpu.sync_copy(pe_hbm, pe_sh)

    plsc.subcore_barrier()

    base = wid * ROWS_PER_W
    obs = (ob0, ob1, ob2, ob3, ob4)
    gsems = (gs0, gs1, gs2, gs3, gs4)
    ssems = (ss0, ss1, ss2, ss3, ss4)
    psems = (ps0, ps1, ps2, ps3, ps4)

    def pe_slice(cc):
        return pe_sh.at[pl.ds((cc * CHUNK) % SEQ, CHUNK)]

    def pe_fill(cc, bb):
        pltpu.async_copy(pe_slice(cc), obs[bb], psems[bb])

    def gather_add(cc, bb):
        pltpu.async_copy(table_hbm.at[idx_v.at[cc]], obs[bb], gsems[bb],
                         add=True)

    def out_slice(cc):
        return out_hbm.at[pl.ds(base + cc * CHUNK, CHUNK)]

    for b in range(SKEW):
        pe_fill(b, b)
    for b in range(SKEW - 1):
        pltpu.make_async_copy(pe_slice(b), obs[b], psems[b]).wait()
        gather_add(b, b)

    def ring_body(i, carry):
        c = i * NBUF
        for b in range(NBUF):
            cc = c + b
            o_v, gs, ss = obs[b], gsems[b], ssems[b]
            pltpu.make_async_copy(table_hbm.at[idx_v.at[cc]], o_v, gs).wait()
            pltpu.async_copy(o_v, out_slice(cc), ss)

            @pl.when(cc + SKEW < NCHUNK)
            def _():
                b3 = (b + SKEW) % NBUF

                @pl.when(cc + SKEW >= NBUF)
                def _():
                    pltpu.make_async_copy(obs[b3], out_slice(cc),
                                          ssems[b3]).wait()

                pe_fill(cc + SKEW, b3)

            @pl.when(cc + SKEW - 1 < NCHUNK)
            def _():
                b2 = (b + SKEW - 1) % NBUF
                pltpu.make_async_copy(pe_slice(cc), obs[b2],
                                      psems[b2]).wait()
                gather_add(cc + SKEW - 1, b2)
        return carry

    lax.fori_loop(0, NCHUNK // NBUF, ring_body, 0)
    for b in range(NBUF):
        pltpu.make_async_copy(obs[b], out_hbm.at[pl.ds(base, CHUNK)],
                              ssems[b]).wait()


@jax.jit
def kernel(inputs, tok_emb_table):
    idx3 = inputs.reshape(NW, NCHUNK, CHUNK).astype(jnp.int32)
    pe1 = _pos_table()
    pe = jnp.concatenate([pe1, pe1], axis=0)
    mesh = plsc.VectorSubcoreMesh(core_axis_name="c", subcore_axis_name="s")
    run = functools.partial(
        pl.kernel,
        mesh=mesh,
        out_type=jax.ShapeDtypeStruct((ROWS, MODEL_DIM), jnp.float32),
        scratch_types=[
            pltpu.VMEM((NCHUNK, CHUNK), jnp.int32),
            pltpu.VMEM_SHARED((2 * SEQ, MODEL_DIM), jnp.float32),
        ] + [pltpu.VMEM((CHUNK, MODEL_DIM), jnp.float32)] * NBUF
          + [pltpu.SemaphoreType.DMA] * (3 * NBUF),
    )(_sc_body)
    out = run(idx3, pe, tok_emb_table)
    return out.reshape(BATCH, SEQ, MODEL_DIM)
